# baseline (speedup 1.0000x reference)
import jax
import jax.numpy as jnp
from jax import lax
from jax.experimental import pallas as pl
from jax.experimental.pallas import tpu as pltpu

_H1 = 64
_H2 = 128
_PW = 128


def _mlp_body(x_ref, w1_ref, b1_ref, w2_ref, b2_ref, w3_ref, b3_ref, p_ref):
    h1 = jnp.dot(x_ref[...], w1_ref[...], preferred_element_type=jnp.float32)
    h1 = jnp.maximum(h1 + b1_ref[...], 0.0)
    h2 = jnp.dot(h1, w2_ref[...], preferred_element_type=jnp.float32)
    h2 = jnp.maximum(h2 + b2_ref[...], 0.0)
    h3 = jnp.dot(h2, w3_ref[...], preferred_element_type=jnp.float32)
    p_ref[...] = jnp.maximum(h3 + b3_ref[...], 0.0)


def _corr_body(pr_ref, pc_ref, corr_ref):
    pr = pr_ref[...]
    pc = pc_ref[...]
    sq_r = jnp.sum(pr * pr, axis=-1, keepdims=True)
    sq_c = jnp.sum(pc * pc, axis=-1)
    gram = lax.dot_general(pr, pc, (((1,), (1,)), ((), ())),
                           preferred_element_type=jnp.float32)
    d = jnp.maximum(sq_r + sq_c[None, :] - 2.0 * gram, 0.0)
    corr_ref[...] = jnp.exp(-d)


def kernel(x, w1, b1, w2, b2, w3, b3):
    B, n_in = x.shape
    n_out = w3.shape[1]
    TB = 512
    assert B % TB == 0
    T = B // TB

    w3p = jnp.zeros((w3.shape[0], _PW), jnp.float32).at[:, :n_out].set(w3)
    b3p = jnp.zeros((1, _PW), jnp.float32).at[:, :n_out].set(b3)

    p_full = pl.pallas_call(
        _mlp_body,
        out_shape=jax.ShapeDtypeStruct((B, _PW), jnp.float32),
        grid=(T,),
        in_specs=[
            pl.BlockSpec((TB, n_in), lambda i: (i, 0)),
            pl.BlockSpec((n_in, _H1), lambda i: (0, 0)),
            pl.BlockSpec((1, _H1), lambda i: (0, 0)),
            pl.BlockSpec((_H1, _H2), lambda i: (0, 0)),
            pl.BlockSpec((1, _H2), lambda i: (0, 0)),
            pl.BlockSpec((_H2, _PW), lambda i: (0, 0)),
            pl.BlockSpec((1, _PW), lambda i: (0, 0)),
        ],
        out_specs=pl.BlockSpec((TB, _PW), lambda i: (i, 0)),
        compiler_params=pltpu.CompilerParams(
            dimension_semantics=("parallel",)),
        name="spacenet_mlp",
    )(x, w1, b1, w2, b2, w3p, b3p)

    def _fold(gi, gk):
        upper = gk < T - gi
        bi = jnp.where(upper, gi, T - 1 - gi)
        bj = jnp.where(upper, gi + gk, gk - 1)
        return bi, bj

    corr_full = pl.pallas_call(
        _corr_body,
        out_shape=jax.ShapeDtypeStruct((B, B), jnp.float32),
        grid=(T // 2, T + 1),
        in_specs=[
            pl.BlockSpec((TB, _PW), lambda gi, gk: (_fold(gi, gk)[0], 0)),
            pl.BlockSpec((TB, _PW), lambda gi, gk: (_fold(gi, gk)[1], 0)),
        ],
        out_specs=pl.BlockSpec((TB, TB), lambda gi, gk: _fold(gi, gk)),
        compiler_params=pltpu.CompilerParams(
            dimension_semantics=("parallel", "arbitrary")),
        name="spacenet_corr",
    )(p_full, p_full)

    corr = _pack_upper_triangle_dma(corr_full)
    return corr, p_full[:, :n_out]


def _pack_upper_triangle(corr_full):
    B = corr_full.shape[0]
    M = B * (B - 1) // 2
    q = 2 * B - 1
    CH = 128

    def off(r):
        return (r * (q - r)) // 2

    def row_of(m):
        disc = (q * q - 8 * m).astype(jnp.float32)
        r = ((q - jnp.sqrt(disc)) * 0.5).astype(jnp.int32)
        r = jnp.clip(r, 0, B - 2)
        r = jnp.where(m < off(r), r - 1, r)
        r = jnp.where(m < off(r), r - 1, r)
        r = jnp.where(m >= off(r + 1), r + 1, r)
        r = jnp.where(m >= off(r + 1), r + 1, r)
        return r

    corr1d = corr_full.reshape(-1)

    R_T = B - 160
    m_tail0 = (off(R_T) // CH) * CH
    nchunks = m_tail0 // CH

    m0 = jnp.arange(nchunks, dtype=jnp.int32) * CH
    r = row_of(m0)
    in_row = m0 - off(r)
    sa = r * B + r + 1 + in_row
    cut = off(r) + (B - 1 - r) - m0
    sb = (r + 1) * B + (r + 2) - cut
    sb = jnp.clip(sb, 0, B * B - CH)
    sa = jnp.clip(sa, 0, B * B - CH)

    slice128 = jax.vmap(lambda s: lax.dynamic_slice(corr1d, (s,), (CH,)))
    ga = slice128(sa)
    gb = slice128(sb)
    lane = jnp.arange(CH, dtype=jnp.int32)[None, :]
    main = jnp.where(lane < jnp.minimum(cut, CH)[:, None], ga, gb).reshape(-1)

    mt = jnp.arange(m_tail0, M, dtype=jnp.int32)
    rt = row_of(mt)
    jt = mt - off(rt) + rt + 1
    tail = corr1d[rt * B + jt]
    return jnp.concatenate([main, tail])


def _pack_upper_triangle_dma(corr_full):
    B = corr_full.shape[0]
    M = B * (B - 1) // 2
    CH = 128
    q = 2 * B - 1

    def off_i(r):
        return (r * (q - r)) // 2

    R_T = B - 160
    m_tail0 = (off_i(R_T) // CH) * CH
    n_tail = M - m_tail0

    mt = jnp.arange(m_tail0, M, dtype=jnp.int32)
    disc = (q * q - 8 * mt).astype(jnp.float32)
    rt = ((q - jnp.sqrt(disc)) * 0.5).astype(jnp.int32)
    rt = jnp.clip(rt, 0, B - 2)
    rt = jnp.where(mt < off_i(rt), rt - 1, rt)
    rt = jnp.where(mt < off_i(rt), rt - 1, rt)
    rt = jnp.where(mt >= off_i(rt + 1), rt + 1, rt)
    rt = jnp.where(mt >= off_i(rt + 1), rt + 1, rt)
    jt = mt - off_i(rt) + rt + 1
    tail_rows = corr_full[R_T - 1:, :]
    tail_vals = tail_rows.reshape(-1)[(rt - (R_T - 1)) * B + jt]

    LK = 16
    NW = 74
    NBUF = 80
    NST = 72

    def body(src_ref, tail_ref, out_ref, buf2, stg2, sem_src, sem_out,
             sem_tail):
        def a_of(r):
            soff = r * B + r + 1
            return jnp.maximum((soff // CH) * CH - CH, 0)

        def src_copy(r):
            slot = pl.multiple_of((r % LK) * NBUF, 8)
            return pltpu.make_async_copy(
                src_ref.at[pl.ds(a_of(r) // CH, NW), :],
                buf2.at[pl.ds(slot, NW), :],
                sem_src.at[r % LK])

        def run_copy(r):
            d0 = off_i(r)
            bc = (d0 // CH) * CH
            i_hi = ((d0 + B - 1 - r) // CH) * CH
            stg_slot = pl.multiple_of((r % LK) * NST, 8)
            nc = i_hi // CH - bc // CH
            return pltpu.make_async_copy(
                stg2.at[pl.ds(stg_slot, nc), :],
                out_ref.at[pl.ds(bc // CH, nc), :],
                sem_out.at[r % LK])

        for r0 in range(LK - 1):
            src_copy(r0).start()

        lane = lax.broadcasted_iota(jnp.int32, (NST, CH), 1)
        row0 = lax.broadcasted_iota(jnp.int32, (NST, CH), 0) == 0

        def step(r, carry):
            src_copy(r).wait()

            @pl.when(r >= LK)
            def _():
                run_copy(r - LK).wait()

            soff = r * B + r + 1
            d0 = off_i(r)
            bc = (d0 // CH) * CH
            h = d0 - bc
            s0 = (soff - a_of(r)) - h
            lane_sh = s0 % CH
            row_sh = s0 // CH

            vrows = pl.multiple_of((r % LK) * NBUF, 8)
            v2 = buf2[pl.ds(vrows, NBUF), :]
            rolled = pltpu.roll(v2, -lane_sh, axis=1)
            va = rolled[0:NST]
            vb = rolled[1:NST + 1]
            vc = rolled[2:NST + 2]
            wrap = lane >= CH - lane_sh
            w = jnp.where(row_sh == 0,
                          jnp.where(wrap, vb, va),
                          jnp.where(wrap, vc, vb))

            prow = jnp.maximum(
                ((r + LK - 1) % LK) * NBUF + (r * B - CH - a_of(r - 1)) // CH,
                0)
            pblk = buf2[pl.ds(pl.multiple_of((prow >> 3) << 3, 8), 8), :]
            pv = pltpu.roll(pblk, -(prow & 7), axis=0)[0:1]
            pv_r = pltpu.roll(pv, h, axis=1)
            run = jnp.where(row0 & (lane < h), pv_r, w)

            srows = pl.multiple_of((r % LK) * NST, 8)
            stg2[pl.ds(srows, NST), :] = run

            run_copy(r).start()

            @pl.when(r + LK - 1 < R_T)
            def _():
                src_copy(r + LK - 1).start()

            return carry

        lax.fori_loop(0, R_T, step, 0, unroll=False)

        tail_dst = out_ref.at[pl.ds(m_tail0 // CH, n_tail // CH), :]
        pltpu.make_async_copy(tail_ref, tail_dst, sem_tail).start()
        pltpu.make_async_copy(tail_ref, tail_dst, sem_tail).wait()

        def drain(i, carry):
            run_copy(R_T - LK + i).wait()
            return carry

        lax.fori_loop(0, LK, drain, 0, unroll=False)

    out = pl.pallas_call(
        body,
        out_shape=jax.ShapeDtypeStruct((M // CH, CH), jnp.float32),
        in_specs=[pl.BlockSpec(memory_space=pl.MemorySpace.ANY),
                  pl.BlockSpec(memory_space=pltpu.MemorySpace.VMEM)],
        out_specs=pl.BlockSpec(memory_space=pl.MemorySpace.ANY),
        scratch_shapes=[
            pltpu.VMEM((LK * NBUF, CH), jnp.float32),
            pltpu.VMEM((LK * NST, CH), jnp.float32),
            pltpu.SemaphoreType.DMA((LK,)),
            pltpu.SemaphoreType.DMA((LK,)),
            pltpu.SemaphoreType.DMA,
        ],
        name="spacenet_pack",
    )(corr_full.reshape(B * B // CH, CH),
      tail_vals.reshape(n_tail // CH, CH))
    return out.reshape(M)


# speedup vs baseline: 1.2059x; 1.2059x over previous
import jax
import jax.numpy as jnp
from jax import lax
from jax.experimental import pallas as pl
from jax.experimental.pallas import tpu as pltpu

_H1 = 64
_H2 = 128
_PW = 128


def _mlp_body(x_ref, w1_ref, b1_ref, w2_ref, b2_ref, w3_ref, b3_ref, p_ref):
    h1 = jnp.dot(x_ref[...], w1_ref[...], preferred_element_type=jnp.float32)
    h1 = jnp.maximum(h1 + b1_ref[...], 0.0)
    h2 = jnp.dot(h1, w2_ref[...], preferred_element_type=jnp.float32)
    h2 = jnp.maximum(h2 + b2_ref[...], 0.0)
    h3 = jnp.dot(h2, w3_ref[...], preferred_element_type=jnp.float32)
    p_ref[...] = jnp.maximum(h3 + b3_ref[...], 0.0)


def _corr_body(pr_ref, pc_ref, corr_ref):
    pr = pr_ref[...]
    pc = pc_ref[...]
    sq_r = jnp.sum(pr * pr, axis=-1, keepdims=True)
    sq_c = jnp.sum(pc * pc, axis=-1)
    gram = lax.dot_general(pr, pc, (((1,), (1,)), ((), ())),
                           preferred_element_type=jnp.float32)
    d = jnp.maximum(sq_r + sq_c[None, :] - 2.0 * gram, 0.0)
    corr_ref[...] = jnp.exp(-d)


def kernel(x, w1, b1, w2, b2, w3, b3):
    B, n_in = x.shape
    n_out = w3.shape[1]
    TB = 512
    assert B % TB == 0
    T = B // TB

    w3p = jnp.zeros((w3.shape[0], _PW), jnp.float32).at[:, :n_out].set(w3)
    b3p = jnp.zeros((1, _PW), jnp.float32).at[:, :n_out].set(b3)

    p_full = pl.pallas_call(
        _mlp_body,
        out_shape=jax.ShapeDtypeStruct((B, _PW), jnp.float32),
        grid=(T,),
        in_specs=[
            pl.BlockSpec((TB, n_in), lambda i: (i, 0)),
            pl.BlockSpec((n_in, _H1), lambda i: (0, 0)),
            pl.BlockSpec((1, _H1), lambda i: (0, 0)),
            pl.BlockSpec((_H1, _H2), lambda i: (0, 0)),
            pl.BlockSpec((1, _H2), lambda i: (0, 0)),
            pl.BlockSpec((_H2, _PW), lambda i: (0, 0)),
            pl.BlockSpec((1, _PW), lambda i: (0, 0)),
        ],
        out_specs=pl.BlockSpec((TB, _PW), lambda i: (i, 0)),
        compiler_params=pltpu.CompilerParams(
            dimension_semantics=("parallel",)),
        name="spacenet_mlp",
    )(x, w1, b1, w2, b2, w3p, b3p)

    def _fold(gi, gk):
        upper = gk < T - gi
        bi = jnp.where(upper, gi, T - 1 - gi)
        bj = jnp.where(upper, gi + gk, gk - 1)
        return bi, bj

    corr_full = pl.pallas_call(
        _corr_body,
        out_shape=jax.ShapeDtypeStruct((B, B), jnp.float32),
        grid=(T // 2, T + 1),
        in_specs=[
            pl.BlockSpec((TB, _PW), lambda gi, gk: (_fold(gi, gk)[0], 0)),
            pl.BlockSpec((TB, _PW), lambda gi, gk: (_fold(gi, gk)[1], 0)),
        ],
        out_specs=pl.BlockSpec((TB, TB), lambda gi, gk: _fold(gi, gk)),
        compiler_params=pltpu.CompilerParams(
            dimension_semantics=("parallel", "arbitrary")),
        name="spacenet_corr",
    )(p_full, p_full)

    corr = _pack_upper_triangle_dma(corr_full)
    return corr, p_full[:, :n_out]


def _pack_upper_triangle(corr_full):
    B = corr_full.shape[0]
    M = B * (B - 1) // 2
    q = 2 * B - 1
    CH = 128

    def off(r):
        return (r * (q - r)) // 2

    def row_of(m):
        disc = (q * q - 8 * m).astype(jnp.float32)
        r = ((q - jnp.sqrt(disc)) * 0.5).astype(jnp.int32)
        r = jnp.clip(r, 0, B - 2)
        r = jnp.where(m < off(r), r - 1, r)
        r = jnp.where(m < off(r), r - 1, r)
        r = jnp.where(m >= off(r + 1), r + 1, r)
        r = jnp.where(m >= off(r + 1), r + 1, r)
        return r

    corr1d = corr_full.reshape(-1)

    R_T = B - 160
    m_tail0 = (off(R_T) // CH) * CH
    nchunks = m_tail0 // CH

    m0 = jnp.arange(nchunks, dtype=jnp.int32) * CH
    r = row_of(m0)
    in_row = m0 - off(r)
    sa = r * B + r + 1 + in_row
    cut = off(r) + (B - 1 - r) - m0
    sb = (r + 1) * B + (r + 2) - cut
    sb = jnp.clip(sb, 0, B * B - CH)
    sa = jnp.clip(sa, 0, B * B - CH)

    slice128 = jax.vmap(lambda s: lax.dynamic_slice(corr1d, (s,), (CH,)))
    ga = slice128(sa)
    gb = slice128(sb)
    lane = jnp.arange(CH, dtype=jnp.int32)[None, :]
    main = jnp.where(lane < jnp.minimum(cut, CH)[:, None], ga, gb).reshape(-1)

    mt = jnp.arange(m_tail0, M, dtype=jnp.int32)
    rt = row_of(mt)
    jt = mt - off(rt) + rt + 1
    tail = corr1d[rt * B + jt]
    return jnp.concatenate([main, tail])


def _pack_upper_triangle_dma(corr_full):
    B = corr_full.shape[0]
    M = B * (B - 1) // 2
    CH = 128
    q = 2 * B - 1

    def off_i(r):
        return (r * (q - r)) // 2

    R_T = B - 160
    m_tail0 = (off_i(R_T) // CH) * CH
    n_tail = M - m_tail0

    mt = jnp.arange(m_tail0, M, dtype=jnp.int32)
    disc = (q * q - 8 * mt).astype(jnp.float32)
    rt = ((q - jnp.sqrt(disc)) * 0.5).astype(jnp.int32)
    rt = jnp.clip(rt, 0, B - 2)
    rt = jnp.where(mt < off_i(rt), rt - 1, rt)
    rt = jnp.where(mt < off_i(rt), rt - 1, rt)
    rt = jnp.where(mt >= off_i(rt + 1), rt + 1, rt)
    rt = jnp.where(mt >= off_i(rt + 1), rt + 1, rt)
    jt = mt - off_i(rt) + rt + 1
    tail_rows = corr_full[R_T - 1:, :]
    tail_vals = tail_rows.reshape(-1)[(rt - (R_T - 1)) * B + jt]

    LK = 16
    NRC = 66 * CH
    NST = 65 * CH
    NV = NRC // CH

    def body(src_ref, tail_ref, out_ref, buf, stg, sem_src, sem_out,
             sem_tail):
        def src_copy(r):
            soff = r * B + r + 1
            a = pl.multiple_of((soff // CH) * CH, CH)
            slot = pl.multiple_of((r % LK) * NRC, CH)
            return pltpu.make_async_copy(
                src_ref.at[pl.ds(a, NRC)], buf.at[pl.ds(slot, NRC)],
                sem_src.at[r % LK])

        def run_copy(r):
            d0 = off_i(r)
            bc = (d0 // CH) * CH
            i_hi = ((d0 + B - 1 - r) // CH) * CH
            stg_slot = pl.multiple_of((r % LK) * NST, CH)
            sz = pl.multiple_of(i_hi - bc, CH)
            return pltpu.make_async_copy(
                stg.at[pl.ds(stg_slot, sz)],
                out_ref.at[pl.ds(pl.multiple_of(bc, CH), sz)],
                sem_out.at[r % LK])

        for r0 in range(LK - 1):
            src_copy(r0).start()

        lane = lax.broadcasted_iota(jnp.int32, (1, CH), 1)
        lane64 = lax.broadcasted_iota(jnp.int32, (NV - 2, CH), 1)

        def step(r, carry):
            src_copy(r).wait()

            @pl.when(r >= LK)
            def _():
                run_copy(r - LK).wait()

            soff = r * B + r + 1
            d0 = off_i(r)
            d1 = d0 + (B - 1 - r)
            bc = (d0 // CH) * CH
            i_hi = (d1 // CH) * CH
            hs = soff - (soff // CH) * CH
            h = d0 - bc

            slot = pl.multiple_of((r % LK) * NRC, CH)
            v = buf[pl.ds(slot, NRC)]
            v2 = v.reshape(NV, CH)

            s0 = hs + CH - h
            lane_sh = s0 % CH
            row_sh = s0 // CH
            rolled = pltpu.roll(v2, -lane_sh, axis=1)
            va = rolled[0:NV - 2]
            vb = rolled[1:NV - 1]
            vc = rolled[2:NV]
            wrap = lane64 >= CH - lane_sh
            w = jnp.where(row_sh == 0,
                          jnp.where(wrap, vb, va),
                          jnp.where(wrap, vc, vb))
            stg_slot = pl.multiple_of((r % LK) * NST, CH)
            stg[pl.ds(stg_slot + CH, NST - CH)] = w.reshape(NST - CH)

            rm = r % CH
            pv_off = pl.multiple_of(
                ((r + LK - 1) % LK) * NRC + B - CH - (r - rm), CH)
            pv = buf[pl.ds(pv_off, CH)].reshape(1, CH)
            pv_r = pltpu.roll(pv, h, axis=1)
            v256 = v[0:2 * CH].reshape(1, 2 * CH)
            hv = pltpu.roll(v256, (h - hs) % (2 * CH), axis=1)
            chunk = jnp.where(lane < h, pv_r, hv[:, 0:CH])
            stg[pl.ds(stg_slot, CH)] = chunk.reshape(CH)

            run_copy(r).start()

            @pl.when(r + LK - 1 < R_T)
            def _():
                src_copy(r + LK - 1).start()

            return carry

        def step2(g, carry):
            step(2 * g, carry)
            step(2 * g + 1, carry)
            return carry

        lax.fori_loop(0, R_T // 2, step2, 0, unroll=False)

        pltpu.make_async_copy(tail_ref, out_ref.at[pl.ds(m_tail0, n_tail)],
                              sem_tail).start()
        pltpu.make_async_copy(tail_ref, out_ref.at[pl.ds(m_tail0, n_tail)],
                              sem_tail).wait()

        def drain(i, carry):
            run_copy(R_T - LK + i).wait()
            return carry

        lax.fori_loop(0, LK, drain, 0, unroll=False)

    return pl.pallas_call(
        body,
        out_shape=jax.ShapeDtypeStruct((M,), jnp.float32),
        in_specs=[pl.BlockSpec(memory_space=pl.MemorySpace.ANY),
                  pl.BlockSpec(memory_space=pltpu.MemorySpace.VMEM)],
        out_specs=pl.BlockSpec(memory_space=pl.MemorySpace.ANY),
        scratch_shapes=[
            pltpu.VMEM((LK * NRC,), jnp.float32),
            pltpu.VMEM((LK * NST,), jnp.float32),
            pltpu.SemaphoreType.DMA((LK,)),
            pltpu.SemaphoreType.DMA((LK,)),
            pltpu.SemaphoreType.DMA,
        ],
        name="spacenet_pack",
    )(corr_full.reshape(-1), tail_vals)
